# Initial kernel scaffold; baseline (speedup 1.0000x reference)
import jax
import jax.numpy as jnp
from jax.experimental import pallas as pl


def kernel(xyz, points, W, b, gamma, beta):
    raise NotImplementedError("write your pallas kernel here")


# speedup vs baseline: 19.8481x; 19.8481x over previous
import functools

import jax
import jax.numpy as jnp
from jax import lax
from jax.experimental import pallas as pl
from jax.experimental.pallas import tpu as pltpu
from jax.experimental.pallas import tpu_sc as plsc

B = 2
N = 8192
S = 2048
K = 16
CIN = 128
CO = 256
CF = 144
TS_KNN = 256
TS_MLP = 128
NWORK = 32
ROWS_PER_GATHER = 128


def _fps_body(xyz_ref, nx_ref):
    x0 = xyz_ref[:, 0]
    x1 = xyz_ref[:, 1]
    x2 = xyz_ref[:, 2]
    rows = x0.shape[1]
    cols = x0.shape[2]
    gidx = (lax.broadcasted_iota(jnp.int32, (B, rows, cols), 1) * cols
            + lax.broadcasted_iota(jnp.int32, (B, rows, cols), 2))

    def body(i, state):
        dists, far = state
        sel = gidx == far
        c0 = jnp.sum(jnp.where(sel, x0, 0.0), axis=(1, 2), keepdims=True)
        c1 = jnp.sum(jnp.where(sel, x1, 0.0), axis=(1, 2), keepdims=True)
        c2 = jnp.sum(jnp.where(sel, x2, 0.0), axis=(1, 2), keepdims=True)
        nx_ref[:, pl.ds(i, 1), :] = jnp.concatenate([c0, c1, c2], axis=2)
        d = ((x0 - c0) ** 2 + (x1 - c1) ** 2) + (x2 - c2) ** 2
        dists = jnp.minimum(dists, d)
        m = jnp.max(dists, axis=(1, 2), keepdims=True)
        far2 = jnp.min(jnp.where(dists == m, gidx, N), axis=(1, 2), keepdims=True)
        return dists, far2

    dists0 = jnp.full((B, rows, cols), 1e10, dtype=jnp.float32)
    far0 = jnp.zeros((B, 1, 1), dtype=jnp.int32)
    lax.fori_loop(0, S, body, (dists0, far0))


def _fps(xyz):
    xyzf = xyz.reshape(B, 3, 8, N // 8)
    return pl.pallas_call(
        _fps_body,
        out_shape=jax.ShapeDtypeStruct((B, S, 3), jnp.float32),
    )(xyzf)


def _knn_body(q_ref, xyz_ref, idx_ref):
    b = pl.program_id(0)
    p0 = xyz_ref[0, 0]
    p1 = xyz_ref[0, 1]
    p2c = xyz_ref[0, 2]
    psq = ((p0 ** 2 + p1 ** 2) + p2c ** 2)[None, :]
    q = q_ref[0]
    qsq = ((q[:, 0] ** 2 + q[:, 1] ** 2) + q[:, 2] ** 2)[:, None]
    bq = q.astype(jnp.bfloat16)
    bp = xyz_ref[0].astype(jnp.bfloat16)
    e = lax.dot_general(bq, bp, (((1,), (0,)), ((), ())),
                        preferred_element_type=jnp.float32)
    dd = (qsq + psq) - 2.0 * e
    iota = lax.broadcasted_iota(jnp.int32, (TS_KNN, N), 1)
    base = b * N
    inf = jnp.float32(jnp.inf)

    for k in range(K):
        m = jnp.min(dd, axis=1, keepdims=True)
        pos = jnp.min(jnp.where(dd == m, iota, N), axis=1, keepdims=True)
        idx_ref[0, :, k] = (pos + base)[:, 0]
        dd = jnp.where(iota == pos, inf, dd)


def _knn(new_xyz_t, xyz):
    return pl.pallas_call(
        _knn_body,
        grid=(B, S // TS_KNN),
        in_specs=[
            pl.BlockSpec((1, TS_KNN, 3), lambda b, j: (b, j, 0)),
            pl.BlockSpec((1, 3, N), lambda b, j: (b, 0, 0)),
        ],
        out_specs=pl.BlockSpec((1, TS_KNN, K), lambda b, j: (b, j, 0)),
        out_shape=jax.ShapeDtypeStruct((B, S, K), jnp.int32),
        compiler_params=pltpu.CompilerParams(
            dimension_semantics=("parallel", "parallel")),
    )(new_xyz_t, xyz)


def _gather_sc(table, idx2d):
    total = B * S * K
    per_w = total // NWORK
    n_ch = per_w // ROWS_PER_GATHER
    mesh = plsc.VectorSubcoreMesh(core_axis_name="c", subcore_axis_name="s")

    @functools.partial(
        pl.kernel,
        mesh=mesh,
        out_type=jax.ShapeDtypeStruct((total, CF), jnp.float32),
        scratch_types=[
            pltpu.VMEM((n_ch, ROWS_PER_GATHER), jnp.int32),
            pltpu.VMEM((ROWS_PER_GATHER, CF), jnp.float32),
            pltpu.VMEM((ROWS_PER_GATHER, CF), jnp.float32),
            pltpu.SemaphoreType.DMA,
            pltpu.SemaphoreType.DMA,
        ],
        compiler_params=pltpu.CompilerParams(use_tc_tiling_on_sc=False),
    )
    def k(table_hbm, idx_hbm, out_hbm, idx_v, rows_a, rows_b, sem_a, sem_b):
        wid = lax.axis_index("s") * 2 + lax.axis_index("c")
        base = wid * per_w
        pltpu.sync_copy(idx_hbm.at[pl.ds(wid * n_ch, n_ch)], idx_v)
        bufs = ((rows_a, sem_a), (rows_b, sem_b))

        def issue(j):
            buf, sem = bufs[j % 2]
            return pltpu.async_copy(table_hbm.at[idx_v.at[j]], buf, sem), buf

        pending = {0: issue(0)}
        for j in range(n_ch):
            cp, buf = pending.pop(j)
            if j + 1 < n_ch:
                pending[j + 1] = issue(j + 1)
            cp.wait()
            pltpu.sync_copy(buf, out_hbm.at[pl.ds(base + j * ROWS_PER_GATHER,
                                                  ROWS_PER_GATHER)])

    return k(table, idx2d)


def _mlp_body(gf_ref, nx_ref, w_ref, b_ref, hmax_ref, hmin_ref, sums_ref):
    j = pl.program_id(1)
    X = gf_ref[0]
    corr = nx_ref[0]
    sub3 = jnp.concatenate([corr] * K, axis=0)
    zeros_l = jnp.zeros((TS_MLP * K, CIN), jnp.float32)
    zeros_r = jnp.zeros((TS_MLP * K, CF - CIN - 3), jnp.float32)
    sub = jnp.concatenate([zeros_l, sub3, zeros_r], axis=1)
    Y = (X - sub).astype(jnp.bfloat16)
    h = lax.dot_general(Y, w_ref[...], (((1,), (0,)), ((), ())),
                        preferred_element_type=jnp.float32)
    h = h + b_ref[...]
    hmax = h[0:TS_MLP]
    hmin = h[0:TS_MLP]
    for k in range(1, K):
        blk = h[k * TS_MLP:(k + 1) * TS_MLP]
        hmax = jnp.maximum(hmax, blk)
        hmin = jnp.minimum(hmin, blk)
    hmax_ref[0] = hmax
    hmin_ref[0] = hmin
    s1 = jnp.sum(h, axis=0, keepdims=True)
    s2 = jnp.sum(h * h, axis=0, keepdims=True)
    part = jnp.concatenate([s1, s2], axis=0)[None]

    @pl.when(j == 0)
    def _():
        sums_ref[...] = jnp.zeros_like(sums_ref)

    sums_ref[...] += part


def _mlp(gf3, nxt, wt_bf, b2):
    return pl.pallas_call(
        _mlp_body,
        grid=(B, S // TS_MLP),
        in_specs=[
            pl.BlockSpec((1, TS_MLP * K, CF), lambda b, j: (b, j, 0)),
            pl.BlockSpec((1, TS_MLP, 3), lambda b, j: (b, j, 0)),
            pl.BlockSpec((CF, CO), lambda b, j: (0, 0)),
            pl.BlockSpec((1, CO), lambda b, j: (0, 0)),
        ],
        out_specs=[
            pl.BlockSpec((1, TS_MLP, CO), lambda b, j: (b, j, 0)),
            pl.BlockSpec((1, TS_MLP, CO), lambda b, j: (b, j, 0)),
            pl.BlockSpec((1, 2, CO), lambda b, j: (b, 0, 0)),
        ],
        out_shape=[
            jax.ShapeDtypeStruct((B, S, CO), jnp.float32),
            jax.ShapeDtypeStruct((B, S, CO), jnp.float32),
            jax.ShapeDtypeStruct((B, 2, CO), jnp.float32),
        ],
        compiler_params=pltpu.CompilerParams(
            dimension_semantics=("parallel", "arbitrary")),
    )(gf3, nxt, wt_bf, b2)


def _fin_body(hmax_ref, hmin_ref, sums_ref, g_ref, be_ref, out_ref):
    cnt = jnp.float32(B * S * K)
    tot = sums_ref[0] + sums_ref[1]
    mean = (tot[0:1] / cnt)
    ex2 = tot[1:2] / cnt
    var = ex2 - mean * mean
    std = jnp.sqrt(var + 1e-5)
    g = g_ref[...]
    sel = jnp.where(g >= 0.0, hmax_ref[0], hmin_ref[0])
    out_ref[0] = jnp.maximum((sel - mean) / std * g + be_ref[...], 0.0)


def _finalize(hmax, hmin, sums, g2, be2):
    return pl.pallas_call(
        _fin_body,
        grid=(B, S // TS_MLP),
        in_specs=[
            pl.BlockSpec((1, TS_MLP, CO), lambda b, j: (b, j, 0)),
            pl.BlockSpec((1, TS_MLP, CO), lambda b, j: (b, j, 0)),
            pl.BlockSpec((B, 2, CO), lambda b, j: (0, 0, 0)),
            pl.BlockSpec((1, CO), lambda b, j: (0, 0)),
            pl.BlockSpec((1, CO), lambda b, j: (0, 0)),
        ],
        out_specs=pl.BlockSpec((1, TS_MLP, CO), lambda b, j: (b, j, 0)),
        out_shape=jax.ShapeDtypeStruct((B, S, CO), jnp.float32),
        compiler_params=pltpu.CompilerParams(
            dimension_semantics=("parallel", "parallel")),
    )(hmax, hmin, sums, g2, be2)


def kernel(xyz, points, W, b, gamma, beta):
    nxt = _fps(xyz)
    new_xyz = jnp.transpose(nxt, (0, 2, 1))
    idxg = _knn(nxt, xyz)

    idx_perm = idxg.reshape(B, S // TS_MLP, TS_MLP, K).transpose(0, 1, 3, 2)
    idx2d = idx_perm.reshape(-1, ROWS_PER_GATHER)

    pT = jnp.transpose(points, (0, 2, 1))
    xT = jnp.transpose(xyz, (0, 2, 1))
    pad = jnp.zeros((B, N, CF - CIN - 3), jnp.float32)
    table = jnp.concatenate([pT, xT, pad], axis=2).reshape(B * N, CF)

    gf = _gather_sc(table, idx2d)
    gf3 = gf.reshape(B, S * K, CF)

    wt = jnp.concatenate([W, jnp.zeros((CO, CF - CIN - 3), jnp.float32)],
                         axis=1).T.astype(jnp.bfloat16)
    b2 = b.reshape(1, CO)
    hmax, hmin, sums = _mlp(gf3, nxt, wt, b2)
    out = _finalize(hmax, hmin, sums, gamma.reshape(1, CO), beta.reshape(1, CO))
    new_points = jnp.transpose(out, (0, 2, 1))
    return (new_xyz, new_points)
